# baseline (speedup 1.0000x reference)
import functools

import jax
import jax.numpy as jnp
from jax import lax
from jax.experimental import pallas as pl
from jax.experimental.pallas import tpu as pltpu
from jax.experimental.pallas import tpu_sc as plsc

VOCAB = 1000000
H = 64
C = 4
B = 16384
L = 50
N = B * L

NC = 2
NS = 16
NW = NC * NS
ROWS_PER_W = N // NW
CHUNK = 128
CHUNKS_PER_W = ROWS_PER_W // CHUNK

@functools.cache
def _make_sc_gather():
    mesh = plsc.VectorSubcoreMesh(core_axis_name="c", subcore_axis_name="s")

    @functools.partial(
        pl.kernel,
        out_type=jax.ShapeDtypeStruct((N, H), jnp.float32),
        mesh=mesh,
        scratch_types=[
            pltpu.VMEM((CHUNKS_PER_W, CHUNK), jnp.int32),
            pltpu.VMEM((CHUNK, H), jnp.float32),
            pltpu.SemaphoreType.DMA,
        ],
        compiler_params=pltpu.CompilerParams(use_tc_tiling_on_sc=False),
    )
    def _sc_gather(ids_hbm, table_hbm, out_hbm, idx_v, rows_v, sem):
        wid = lax.axis_index("s") * NC + lax.axis_index("c")
        pltpu.sync_copy(ids_hbm.at[wid], idx_v)
        base = wid * ROWS_PER_W

        def step(j, carry):
            pltpu.async_copy(table_hbm.at[idx_v.at[j]], rows_v, sem).wait()
            pltpu.sync_copy(rows_v, out_hbm.at[pl.ds(base + j * CHUNK, CHUNK)])
            return carry

        lax.fori_loop(0, CHUNKS_PER_W, step, 0)

    return _sc_gather


def _mm_body(item_ref, ctx_ref, w1t_ref, wct_ref, be_ref, out_ref):
    acc = jnp.dot(item_ref[...], w1t_ref[...], preferred_element_type=jnp.float32)
    acc += jnp.dot(ctx_ref[...], wct_ref[...], preferred_element_type=jnp.float32)
    out_ref[...] = acc + be_ref[...]


NB = 2048


def _tc_fused(item_rows, ctx, w1t, wct, b_eff):
    grid = (N // NB,)
    return pl.pallas_call(
        _mm_body,
        grid=grid,
        in_specs=[
            pl.BlockSpec((NB, H), lambda i: (i, 0)),
            pl.BlockSpec((NB, C), lambda i: (i, 0)),
            pl.BlockSpec((H, H), lambda i: (0, 0)),
            pl.BlockSpec((C, H), lambda i: (0, 0)),
            pl.BlockSpec((1, H), lambda i: (0, 0)),
        ],
        out_specs=pl.BlockSpec((NB, H), lambda i: (i, 0)),
        out_shape=jax.ShapeDtypeStruct((N, H), jnp.float32),
    )(item_rows, ctx, w1t, wct, b_eff)


def kernel(item_ids, context_features, item_table, W_ctx, b_ctx, W_joint, b_joint):
    ids3 = item_ids.astype(jnp.int32).reshape(NW, CHUNKS_PER_W, CHUNK)
    item_rows = _make_sc_gather()(ids3, item_table)

    W1 = W_joint[:, :H]
    W2 = W_joint[:, H:]
    w1t = W1.T
    wct = (W2 @ W_ctx).T
    b_eff = (b_joint + W2 @ b_ctx).reshape(1, H)

    ctx2 = context_features.reshape(N, C)
    out = _tc_fused(item_rows, ctx2, w1t, wct, b_eff)
    return out.reshape(B, L, H)


# speedup vs baseline: 1.8944x; 1.8944x over previous
import functools

import jax
import jax.numpy as jnp
from jax import lax
from jax.experimental import pallas as pl
from jax.experimental.pallas import tpu as pltpu
from jax.experimental.pallas import tpu_sc as plsc

VOCAB = 1000000
H = 64
C = 4
B = 16384
L = 50
N = B * L
NP = N // 2

NC = 2
NS = 16
NW = NC * NS
ROWS_PER_W = NP // NW
PCHUNK = 64
CHUNKS_PER_W = ROWS_PER_W // PCHUNK

BB = 2048
HALF = BB // 2
NCB = B // BB
PPB = HALF


@functools.cache
def _make_sc_gather():
    mesh = plsc.VectorSubcoreMesh(core_axis_name="c", subcore_axis_name="s")

    @functools.partial(
        pl.kernel,
        out_type=jax.ShapeDtypeStruct((NP, 128), jnp.float32),
        mesh=mesh,
        scratch_types=[
            pltpu.VMEM((CHUNKS_PER_W, 2, PCHUNK), jnp.int32),
            pltpu.VMEM((PCHUNK, H), jnp.float32),
            pltpu.VMEM((PCHUNK, H), jnp.float32),
            pltpu.SemaphoreType.DMA,
        ],
        compiler_params=pltpu.CompilerParams(use_tc_tiling_on_sc=False),
    )
    def _sc_gather(ids_hbm, table_hbm, out_hbm, idx_v, lo_v, hi_v, sem):
        wid = lax.axis_index("s") * NC + lax.axis_index("c")
        pltpu.sync_copy(ids_hbm.at[wid], idx_v)
        base = wid * ROWS_PER_W

        def step(j, carry):
            lo = pltpu.async_copy(table_hbm.at[idx_v.at[j, 0]], lo_v, sem)
            hi = pltpu.async_copy(table_hbm.at[idx_v.at[j, 1]], hi_v, sem)
            lo.wait()
            hi.wait()
            rows = pl.ds(base + j * PCHUNK, PCHUNK)
            pltpu.sync_copy(lo_v, out_hbm.at[rows, pl.ds(0, H)])
            pltpu.sync_copy(hi_v, out_hbm.at[rows, pl.ds(H, H)])
            return carry

        lax.fori_loop(0, CHUNKS_PER_W, step, 0)

    return _sc_gather


def _mm_body(gp_ref, ctx_ref, w1_ref, wct_ref, be_ref, out_ref):
    a = gp_ref[...]
    w1 = w1_ref[...]
    dn = (((1,), (1,)), ((), ()))
    r_lo = lax.dot_general(w1, a[:, :H], dn, preferred_element_type=jnp.float32)
    r_hi = lax.dot_general(w1, a[:, H:], dn, preferred_element_type=jnp.float32)
    rc = jnp.dot(wct_ref[...], ctx_ref[0], preferred_element_type=jnp.float32)
    acc = jnp.concatenate([r_lo, r_hi], axis=1) + rc + be_ref[...]
    out_ref[0] = acc


def _tc_fused(gp, ctx_t, w1, wct_t, b_eff):
    return pl.pallas_call(
        _mm_body,
        grid=(L, NCB),
        in_specs=[
            pl.BlockSpec((PPB, 128), lambda l, c: (l * NCB + c, 0)),
            pl.BlockSpec((1, C, BB), lambda l, c: (l, 0, c)),
            pl.BlockSpec((H, H), lambda l, c: (0, 0)),
            pl.BlockSpec((H, C), lambda l, c: (0, 0)),
            pl.BlockSpec((H, 1), lambda l, c: (0, 0)),
        ],
        out_specs=pl.BlockSpec((1, H, BB), lambda l, c: (l, 0, c)),
        out_shape=jax.ShapeDtypeStruct((L, H, B), jnp.float32),
    )(gp, ctx_t, w1, wct_t, b_eff)


def kernel(item_ids, context_features, item_table, W_ctx, b_ctx, W_joint, b_joint):
    idt = item_ids.astype(jnp.int32).T
    pairs = idt.reshape(L, NCB, 2, HALF)
    pairs = pairs.transpose(0, 1, 3, 2)
    ids4 = pairs.reshape(NW, CHUNKS_PER_W, PCHUNK, 2).transpose(0, 1, 3, 2)

    gp = _make_sc_gather()(ids4, item_table)

    W1 = W_joint[:, :H]
    W2 = W_joint[:, H:]
    wct_t = W2 @ W_ctx
    b_eff = (b_joint + W2 @ b_ctx).reshape(H, 1)

    ctx_t = context_features.transpose(1, 2, 0)
    out_t = _tc_fused(gp, ctx_t, W1, wct_t, b_eff)
    return out_t.transpose(2, 0, 1)
